# Initial kernel scaffold; baseline (speedup 1.0000x reference)
import functools

import jax
import jax.numpy as jnp
from jax import lax
from jax.experimental import pallas as pl
from jax.experimental.pallas import tpu as pltpu
from jax.experimental.pallas import tpu_sc as plsc

N = 8192
B = 8
K = 10
CH = 64
OUT_DIMS = 128
EPS = 1e-9
BN_EPS = 1e-5

ROWS = 256
NBLK = N // ROWS

SC_NC = 2
SC_NS = 16
SC_NW = SC_NC * SC_NS
IDX_TOTAL = N * K
IDX_PER_W = IDX_TOTAL // SC_NW
SC_CHUNK = 640


def _prep_body(f_ref, w_ref, b_ref, p_ref, q_ref):
    pq = jnp.dot(f_ref[...], w_ref[...], preferred_element_type=jnp.float32)
    pq = pq + b_ref[...]
    p_ref[...] = pq[:, :CH]
    q_ref[...] = pq[:, CH:]


def _knn_body(ft_ref, f_ref, idx_ref):
    ft = ft_ref[...]
    sq = jnp.sum(ft * ft, axis=0, keepdims=True)
    d = sq - 2.0 * jnp.dot(f_ref[...], ft, preferred_element_type=jnp.float32)
    lane = lax.broadcasted_iota(jnp.int32, d.shape, 1)
    cols = []
    for _ in range(K):
        m = jnp.min(d, axis=1, keepdims=True)
        cand = jnp.where(d == m, lane, jnp.int32(2**30))
        j = jnp.min(cand, axis=1, keepdims=True)
        cols.append(j)
        d = jnp.where(lane == j, jnp.inf, d)
    pad = jnp.zeros((d.shape[0], 16 - K), jnp.int32)
    idx_ref[...] = jnp.concatenate(cols + [pad], axis=1)


def _bnstats_body(p_ref, qg_ref, g_ref, be_ref, ss_ref, acc):
    i = pl.program_id(0)

    @pl.when(i == 0)
    def _():
        acc[...] = jnp.zeros((8, CH), jnp.float32)

    p = p_ref[...]
    qg = qg_ref[...]
    s = jnp.sum(qg, axis=1)
    t = jnp.sum(qg * qg, axis=1)
    rows = [
        jnp.sum(p, axis=0, keepdims=True),
        jnp.sum(p * p, axis=0, keepdims=True),
        jnp.sum(s, axis=0, keepdims=True),
        jnp.sum(p * s, axis=0, keepdims=True),
        jnp.sum(t, axis=0, keepdims=True),
    ]
    upd = jnp.concatenate(rows + [jnp.zeros((3, CH), jnp.float32)], axis=0)
    acc[...] = acc[...] + upd

    @pl.when(i == NBLK - 1)
    def _():
        a = acc[...]
        inv_nk = 1.0 / (N * K)
        mean = (K * a[0:1, :] + a[2:3, :]) * inv_nk
        eh2 = (K * a[1:2, :] + 2.0 * a[3:4, :] + a[4:5, :]) * inv_nk
        var = eh2 - mean * mean
        scale = g_ref[...] / jnp.sqrt(var + BN_EPS)
        shift = be_ref[...] - mean * scale
        ss_ref[...] = jnp.concatenate(
            [scale, shift, jnp.zeros((6, CH), jnp.float32)], axis=0)


def _apply_body(p_ref, qg_ref, ss_ref, f_ref):
    scale = ss_ref[0:1, :]
    shift = ss_ref[1:2, :]
    qg = qg_ref[...]
    qmax = jnp.max(qg, axis=1)
    qmin = jnp.min(qg, axis=1)
    qe = jnp.where(scale >= 0.0, qmax, qmin)
    h = (p_ref[...] + qe) * scale + shift
    f_ref[...] = jnp.maximum(h, 0.0)


def _head_body(f1_ref, f2_ref, f3_ref, f4_ref, wp_ref, bp_ref, out_ref):
    parts = []
    for r in (f1_ref, f2_ref, f3_ref, f4_ref):
        f = r[...].reshape(B, N // B, CH)
        parts.append(jnp.max(f, axis=1))
    a = jnp.concatenate(parts, axis=1)
    h = jnp.dot(a, wp_ref[...], preferred_element_type=jnp.float32)
    h = jnp.tanh(h + bp_ref[...])
    nrm = jnp.sqrt(jnp.sum(h * h, axis=1, keepdims=True))
    out_ref[...] = h / (nrm + EPS)


def _prep(f, wcat, bcat):
    return pl.pallas_call(
        _prep_body,
        grid=(NBLK,),
        in_specs=[
            pl.BlockSpec((ROWS, CH), lambda i: (i, 0)),
            pl.BlockSpec((CH, 2 * CH), lambda i: (0, 0)),
            pl.BlockSpec((1, 2 * CH), lambda i: (0, 0)),
        ],
        out_specs=[
            pl.BlockSpec((ROWS, CH), lambda i: (i, 0)),
            pl.BlockSpec((ROWS, CH), lambda i: (i, 0)),
        ],
        out_shape=[
            jax.ShapeDtypeStruct((N, CH), jnp.float32),
            jax.ShapeDtypeStruct((N, CH), jnp.float32),
        ],
    )(f, wcat, bcat)


def _knn(ft, f):
    return pl.pallas_call(
        _knn_body,
        grid=(NBLK,),
        in_specs=[
            pl.BlockSpec((CH, N), lambda i: (0, 0)),
            pl.BlockSpec((ROWS, CH), lambda i: (i, 0)),
        ],
        out_specs=pl.BlockSpec((ROWS, 16), lambda i: (i, 0)),
        out_shape=jax.ShapeDtypeStruct((N, 16), jnp.int32),
    )(ft, f)


def _sc_gather(idx_flat, q):
    mesh = plsc.VectorSubcoreMesh(core_axis_name="c", subcore_axis_name="s")

    @functools.partial(
        pl.kernel,
        mesh=mesh,
        out_type=jax.ShapeDtypeStruct((IDX_TOTAL, CH), jnp.float32),
        scratch_types=[
            pltpu.VMEM((SC_CHUNK,), jnp.int32),
            pltpu.VMEM((SC_CHUNK, CH), jnp.float32),
            pltpu.SemaphoreType.DMA,
        ],
    )
    def gather_kernel(idx_hbm, q_hbm, out_hbm, idx_v, rows_v, sem):
        wid = lax.axis_index("s") * SC_NC + lax.axis_index("c")
        base = wid * IDX_PER_W
        for ci in range(IDX_PER_W // SC_CHUNK):
            off = base + ci * SC_CHUNK
            pltpu.sync_copy(idx_hbm.at[pl.ds(off, SC_CHUNK)], idx_v)
            pltpu.async_copy(q_hbm.at[idx_v], rows_v, sem).wait()
            pltpu.sync_copy(rows_v, out_hbm.at[pl.ds(off, SC_CHUNK)])

    return gather_kernel(idx_flat, q)


def _bnstats(p, qg, g, be):
    return pl.pallas_call(
        _bnstats_body,
        grid=(NBLK,),
        in_specs=[
            pl.BlockSpec((ROWS, CH), lambda i: (i, 0)),
            pl.BlockSpec((ROWS, K, CH), lambda i: (i, 0, 0)),
            pl.BlockSpec((1, CH), lambda i: (0, 0)),
            pl.BlockSpec((1, CH), lambda i: (0, 0)),
        ],
        out_specs=pl.BlockSpec((8, CH), lambda i: (0, 0)),
        out_shape=jax.ShapeDtypeStruct((8, CH), jnp.float32),
        scratch_shapes=[pltpu.VMEM((8, CH), jnp.float32)],
    )(p, qg, g, be)


def _apply(p, qg, ss):
    return pl.pallas_call(
        _apply_body,
        grid=(NBLK,),
        in_specs=[
            pl.BlockSpec((ROWS, CH), lambda i: (i, 0)),
            pl.BlockSpec((ROWS, K, CH), lambda i: (i, 0, 0)),
            pl.BlockSpec((8, CH), lambda i: (0, 0)),
        ],
        out_specs=pl.BlockSpec((ROWS, CH), lambda i: (i, 0)),
        out_shape=jax.ShapeDtypeStruct((N, CH), jnp.float32),
    )(p, qg, ss)


def _head(f1, f2, f3, f4, wp, bp):
    return pl.pallas_call(
        _head_body,
        grid=(1,),
        in_specs=[pl.BlockSpec((N, CH), lambda i: (0, 0))] * 4 + [
            pl.BlockSpec((4 * CH, OUT_DIMS), lambda i: (0, 0)),
            pl.BlockSpec((1, OUT_DIMS), lambda i: (0, 0)),
        ],
        out_specs=pl.BlockSpec((B, OUT_DIMS), lambda i: (0, 0)),
        out_shape=jax.ShapeDtypeStruct((B, OUT_DIMS), jnp.float32),
    )(f1, f2, f3, f4, wp, bp)


def _layer(f, wcat, bcat, ss_const, g=None, be=None):
    ft = f.T
    p, q = _prep(f, wcat, bcat)
    idxp = _knn(ft, f)
    idx_flat = idxp[:, :K].reshape(-1)
    qg = _sc_gather(idx_flat, q).reshape(N, K, CH)
    if g is not None:
        ss = _bnstats(p, qg, g.reshape(1, CH), be.reshape(1, CH))
    else:
        ss = ss_const
    return _apply(p, qg, ss)


def _wcat(w, b, d_in):
    wa, wb = w[:d_in], w[d_in:]
    wd = wa - wb
    wcat = jnp.concatenate([wd, wb], axis=1)
    if d_in < CH:
        wcat = jnp.pad(wcat, ((0, CH - d_in), (0, 0)))
    bcat = jnp.concatenate([b, jnp.zeros((CH,), jnp.float32)]).reshape(1, 2 * CH)
    return wcat, bcat


def kernel(x, n_pts, W1, b1, g1, be1, W2, b2, W3, b3, g3, be3, W4, b4, Wp, bp):
    del n_pts
    xp = jnp.pad(x, ((0, 0), (0, CH - x.shape[1])))
    ss_id = jnp.concatenate(
        [jnp.ones((1, CH), jnp.float32), jnp.zeros((7, CH), jnp.float32)], axis=0)

    w1c, b1c = _wcat(W1, b1, 3)
    w2c, b2c = _wcat(W2, b2, CH)
    w3c, b3c = _wcat(W3, b3, CH)
    w4c, b4c = _wcat(W4, b4, CH)

    f1 = _layer(xp, w1c, b1c, ss_id, g1, be1)
    f2 = _layer(f1, w2c, b2c, ss_id)
    f3 = _layer(f2, w3c, b3c, ss_id)
    f4 = _layer(f3, w4c, b4c, ss_id)

    return _head(f1, f2, f3, f4, Wp, bp.reshape(1, OUT_DIMS))


# speedup vs baseline: 6.7938x; 6.7938x over previous
import functools

import jax
import jax.numpy as jnp
from jax import lax
from jax.experimental import pallas as pl
from jax.experimental.pallas import tpu as pltpu
from jax.experimental.pallas import tpu_sc as plsc

N = 8192
B = 8
K = 10
CH = 64
FW = 128
OUT_DIMS = 128
EPS = 1e-9
BN_EPS = 1e-5

ROWS = 256
NBLK = N // ROWS

SC_NC = 2
SC_NS = 16
SC_NW = SC_NC * SC_NS
IDX_TOTAL = N * K
IDX_PER_W = IDX_TOTAL // SC_NW
SC_CHUNK = 640


def _knn_body(ft_ref, f_ref, idx_ref):
    ft = ft_ref[...]
    sq = jnp.sum(ft * ft, axis=0, keepdims=True)
    d = sq - 2.0 * jnp.dot(f_ref[...].astype(jnp.bfloat16),
                           ft.astype(jnp.bfloat16),
                           preferred_element_type=jnp.float32)
    lane = lax.broadcasted_iota(jnp.int32, d.shape, 1)
    cols = []
    for _ in range(K):
        m = jnp.min(d, axis=1, keepdims=True)
        cand = jnp.where(d == m, lane, jnp.int32(2**30))
        j = jnp.min(cand, axis=1, keepdims=True)
        cols.append(j)
        d = jnp.where(lane == j, jnp.inf, d)
    pad = jnp.zeros((d.shape[0], 16 - K), jnp.int32)
    idx_ref[...] = jnp.concatenate(cols + [pad], axis=1)


def _knn(ft, f):
    return pl.pallas_call(
        _knn_body,
        grid=(NBLK,),
        in_specs=[
            pl.BlockSpec((FW, N), lambda i: (0, 0)),
            pl.BlockSpec((ROWS, FW), lambda i: (i, 0)),
        ],
        out_specs=pl.BlockSpec((ROWS, 16), lambda i: (i, 0)),
        out_shape=jax.ShapeDtypeStruct((N, 16), jnp.int32),
    )(ft, f)


def _sc_gather(idx_flat, table):
    mesh = plsc.VectorSubcoreMesh(core_axis_name="c", subcore_axis_name="s")

    @functools.partial(
        pl.kernel,
        mesh=mesh,
        out_type=jax.ShapeDtypeStruct((IDX_TOTAL, FW), jnp.float32),
        scratch_types=[
            pltpu.VMEM((SC_CHUNK,), jnp.int32),
            pltpu.VMEM((SC_CHUNK, FW), jnp.float32),
            pltpu.SemaphoreType.DMA,
        ],
    )
    def gather_kernel(idx_hbm, tab_hbm, out_hbm, idx_v, rows_v, sem):
        wid = lax.axis_index("s") * SC_NC + lax.axis_index("c")
        base = wid * IDX_PER_W
        for ci in range(IDX_PER_W // SC_CHUNK):
            off = base + ci * SC_CHUNK
            pltpu.sync_copy(idx_hbm.at[pl.ds(off, SC_CHUNK)], idx_v)
            pltpu.async_copy(tab_hbm.at[idx_v], rows_v, sem).wait()
            pltpu.sync_copy(rows_v, out_hbm.at[pl.ds(off, SC_CHUNK)])

    return gather_kernel(idx_flat, table)


def _edge_h(f_ref, g_ref, w_ref, b_ref):
    xi = f_ref[...][:, :CH]
    w = w_ref[...].astype(jnp.bfloat16)
    b = b_ref[...]
    hmax = hmin = None
    sh = sh2 = None
    for j in range(K):
        ej = jnp.concatenate([xi, g_ref[:, j, :CH] - xi], axis=1)
        h = jnp.dot(ej.astype(jnp.bfloat16), w,
                    preferred_element_type=jnp.float32) + b
        hmax = h if hmax is None else jnp.maximum(hmax, h)
        hmin = h if hmin is None else jnp.minimum(hmin, h)
        cs = jnp.sum(h, axis=0, keepdims=True)
        cs2 = jnp.sum(h * h, axis=0, keepdims=True)
        sh = cs if sh is None else sh + cs
        sh2 = cs2 if sh2 is None else sh2 + cs2
    return hmax, hmin, sh, sh2


def _edge_bn_body(f_ref, g_ref, w_ref, b_ref, hmax_ref, hmin_ref, sums_ref, acc):
    i = pl.program_id(0)

    @pl.when(i == 0)
    def _():
        acc[...] = jnp.zeros((8, CH), jnp.float32)

    hmax, hmin, sh, sh2 = _edge_h(f_ref, g_ref, w_ref, b_ref)
    hmax_ref[...] = hmax
    hmin_ref[...] = hmin
    acc[...] = acc[...] + jnp.concatenate(
        [sh, sh2, jnp.zeros((6, CH), jnp.float32)], axis=0)

    @pl.when(i == NBLK - 1)
    def _():
        sums_ref[...] = acc[...]


def _edge_plain_body(f_ref, g_ref, w_ref, b_ref, fo_ref):
    hmax, _, _, _ = _edge_h(f_ref, g_ref, w_ref, b_ref)
    f = jnp.maximum(hmax, 0.0)
    fo_ref[...] = jnp.concatenate(
        [f, jnp.zeros((f.shape[0], FW - CH), jnp.float32)], axis=1)


def _edge_bn(f, g3, w, b):
    return pl.pallas_call(
        _edge_bn_body,
        grid=(NBLK,),
        in_specs=[
            pl.BlockSpec((ROWS, FW), lambda i: (i, 0)),
            pl.BlockSpec((ROWS, K, FW), lambda i: (i, 0, 0)),
            pl.BlockSpec((FW, CH), lambda i: (0, 0)),
            pl.BlockSpec((1, CH), lambda i: (0, 0)),
        ],
        out_specs=[
            pl.BlockSpec((ROWS, CH), lambda i: (i, 0)),
            pl.BlockSpec((ROWS, CH), lambda i: (i, 0)),
            pl.BlockSpec((8, CH), lambda i: (0, 0)),
        ],
        out_shape=[
            jax.ShapeDtypeStruct((N, CH), jnp.float32),
            jax.ShapeDtypeStruct((N, CH), jnp.float32),
            jax.ShapeDtypeStruct((8, CH), jnp.float32),
        ],
        scratch_shapes=[pltpu.VMEM((8, CH), jnp.float32)],
    )(f, g3, w, b)


def _edge_plain(f, g3, w, b):
    return pl.pallas_call(
        _edge_plain_body,
        grid=(NBLK,),
        in_specs=[
            pl.BlockSpec((ROWS, FW), lambda i: (i, 0)),
            pl.BlockSpec((ROWS, K, FW), lambda i: (i, 0, 0)),
            pl.BlockSpec((FW, CH), lambda i: (0, 0)),
            pl.BlockSpec((1, CH), lambda i: (0, 0)),
        ],
        out_specs=pl.BlockSpec((ROWS, FW), lambda i: (i, 0)),
        out_shape=jax.ShapeDtypeStruct((N, FW), jnp.float32),
    )(f, g3, w, b)


def _bnapply_body(hmax_ref, hmin_ref, sums_ref, g_ref, be_ref, fo_ref):
    s = sums_ref[...]
    inv_nk = 1.0 / (N * K)
    mean = s[0:1, :] * inv_nk
    var = s[1:2, :] * inv_nk - mean * mean
    g = g_ref[...]
    hsel = jnp.where(g >= 0.0, hmax_ref[...], hmin_ref[...])
    h = (hsel - mean) / jnp.sqrt(var + BN_EPS) * g + be_ref[...]
    f = jnp.maximum(h, 0.0)
    fo_ref[...] = jnp.concatenate(
        [f, jnp.zeros((f.shape[0], FW - CH), jnp.float32)], axis=1)


def _bnapply(hmax, hmin, sums, g, be):
    return pl.pallas_call(
        _bnapply_body,
        grid=(NBLK,),
        in_specs=[
            pl.BlockSpec((ROWS, CH), lambda i: (i, 0)),
            pl.BlockSpec((ROWS, CH), lambda i: (i, 0)),
            pl.BlockSpec((8, CH), lambda i: (0, 0)),
            pl.BlockSpec((1, CH), lambda i: (0, 0)),
            pl.BlockSpec((1, CH), lambda i: (0, 0)),
        ],
        out_specs=pl.BlockSpec((ROWS, FW), lambda i: (i, 0)),
        out_shape=jax.ShapeDtypeStruct((N, FW), jnp.float32),
    )(hmax, hmin, sums, g, be)


def _head_body(f1_ref, f2_ref, f3_ref, f4_ref, wp_ref, bp_ref, out_ref):
    parts = []
    for r in (f1_ref, f2_ref, f3_ref, f4_ref):
        f = r[...].reshape(B, N // B, FW)
        parts.append(jnp.max(f, axis=1)[:, :CH])
    a = jnp.concatenate(parts, axis=1)
    h = jnp.dot(a.astype(jnp.bfloat16),
                wp_ref[...].astype(jnp.bfloat16),
                preferred_element_type=jnp.float32)
    h = jnp.tanh(h + bp_ref[...])
    nrm = jnp.sqrt(jnp.sum(h * h, axis=1, keepdims=True))
    out_ref[...] = h / (nrm + EPS)


def _head(f1, f2, f3, f4, wp, bp):
    return pl.pallas_call(
        _head_body,
        grid=(1,),
        in_specs=[pl.BlockSpec((N, FW), lambda i: (0, 0))] * 4 + [
            pl.BlockSpec((4 * CH, OUT_DIMS), lambda i: (0, 0)),
            pl.BlockSpec((1, OUT_DIMS), lambda i: (0, 0)),
        ],
        out_specs=pl.BlockSpec((B, OUT_DIMS), lambda i: (0, 0)),
        out_shape=jax.ShapeDtypeStruct((B, OUT_DIMS), jnp.float32),
    )(f1, f2, f3, f4, wp, bp)


def _layer(f, w128, bvec, g=None, be=None):
    idxp = _knn(f.T, f)
    idx_flat = idxp[:, :K].reshape(-1)
    gath = _sc_gather(idx_flat, f).reshape(N, K, FW)
    if g is not None:
        hmax, hmin, sums = _edge_bn(f, gath, w128, bvec)
        return _bnapply(hmax, hmin, sums, g.reshape(1, CH), be.reshape(1, CH))
    return _edge_plain(f, gath, w128, bvec)


def _w128(w, d_in):
    out = jnp.zeros((FW, CH), jnp.float32)
    out = out.at[:d_in].set(w[:d_in])
    out = out.at[CH:CH + d_in].set(w[d_in:])
    return out


def kernel(x, n_pts, W1, b1, g1, be1, W2, b2, W3, b3, g3, be3, W4, b4, Wp, bp):
    del n_pts
    xp = jnp.pad(x, ((0, 0), (0, FW - x.shape[1])))

    f1 = _layer(xp, _w128(W1, 3), b1.reshape(1, CH), g1, be1)
    f2 = _layer(f1, _w128(W2, CH), b2.reshape(1, CH))
    f3 = _layer(f2, _w128(W3, CH), b3.reshape(1, CH), g3, be3)
    f4 = _layer(f3, _w128(W4, CH), b4.reshape(1, CH))

    return _head(f1, f2, f3, f4, Wp, bp.reshape(1, OUT_DIMS))


# speedup vs baseline: 7.2232x; 1.0632x over previous
import functools

import jax
import jax.numpy as jnp
from jax import lax
from jax.experimental import pallas as pl
from jax.experimental.pallas import tpu as pltpu
from jax.experimental.pallas import tpu_sc as plsc

N = 8192
B = 8
K = 10
CH = 64
FW = 128
OUT_DIMS = 128
EPS = 1e-9
BN_EPS = 1e-5

ROWS = 256
NBLK = N // ROWS

SC_NC = 2
SC_NS = 16
SC_NW = SC_NC * SC_NS
IDX_TOTAL = N * K
IDX_PER_W = IDX_TOTAL // SC_NW
SC_CHUNK = 640


def _knn_body(ft_ref, f_ref, idx_ref):
    ft = ft_ref[...]
    sq = jnp.sum(ft * ft, axis=0, keepdims=True)
    d = sq - 2.0 * jnp.dot(f_ref[...].astype(jnp.bfloat16),
                           ft.astype(jnp.bfloat16),
                           preferred_element_type=jnp.float32)
    lane = lax.broadcasted_iota(jnp.int32, d.shape, 1)
    cols = []
    for _ in range(K):
        j = jnp.argmin(d, axis=1).astype(jnp.int32)
        j = j.reshape(d.shape[0], 1)
        cols.append(j)
        d = jnp.where(lane == j, jnp.inf, d)
    pad = jnp.zeros((d.shape[0], 16 - K), jnp.int32)
    idx_ref[...] = jnp.concatenate(cols + [pad], axis=1)


def _knn(ft, f):
    return pl.pallas_call(
        _knn_body,
        grid=(NBLK,),
        in_specs=[
            pl.BlockSpec((FW, N), lambda i: (0, 0)),
            pl.BlockSpec((ROWS, FW), lambda i: (i, 0)),
        ],
        out_specs=pl.BlockSpec((ROWS, 16), lambda i: (i, 0)),
        out_shape=jax.ShapeDtypeStruct((N, 16), jnp.int32),
    )(ft, f)


def _sc_gather(idx_flat, table):
    mesh = plsc.VectorSubcoreMesh(core_axis_name="c", subcore_axis_name="s")

    @functools.partial(
        pl.kernel,
        mesh=mesh,
        out_type=jax.ShapeDtypeStruct((IDX_TOTAL, FW), jnp.float32),
        scratch_types=[
            pltpu.VMEM((SC_CHUNK,), jnp.int32),
            pltpu.VMEM((SC_CHUNK, FW), jnp.float32),
            pltpu.SemaphoreType.DMA,
        ],
    )
    def gather_kernel(idx_hbm, tab_hbm, out_hbm, idx_v, rows_v, sem):
        wid = lax.axis_index("s") * SC_NC + lax.axis_index("c")
        base = wid * IDX_PER_W
        for ci in range(IDX_PER_W // SC_CHUNK):
            off = base + ci * SC_CHUNK
            pltpu.sync_copy(idx_hbm.at[pl.ds(off, SC_CHUNK)], idx_v)
            pltpu.async_copy(tab_hbm.at[idx_v], rows_v, sem).wait()
            pltpu.sync_copy(rows_v, out_hbm.at[pl.ds(off, SC_CHUNK)])

    return gather_kernel(idx_flat, table)


def _edge_h(f_ref, g_ref, w_ref, b_ref):
    xi = f_ref[...][:, :CH]
    w = w_ref[...].astype(jnp.bfloat16)
    b = b_ref[...]
    hmax = hmin = None
    sh = sh2 = None
    for j in range(K):
        ej = jnp.concatenate([xi, g_ref[:, j, :CH] - xi], axis=1)
        h = jnp.dot(ej.astype(jnp.bfloat16), w,
                    preferred_element_type=jnp.float32) + b
        hmax = h if hmax is None else jnp.maximum(hmax, h)
        hmin = h if hmin is None else jnp.minimum(hmin, h)
        cs = jnp.sum(h, axis=0, keepdims=True)
        cs2 = jnp.sum(h * h, axis=0, keepdims=True)
        sh = cs if sh is None else sh + cs
        sh2 = cs2 if sh2 is None else sh2 + cs2
    return hmax, hmin, sh, sh2


def _edge_bn_body(f_ref, g_ref, w_ref, b_ref, hmax_ref, hmin_ref, sums_ref, acc):
    i = pl.program_id(0)

    @pl.when(i == 0)
    def _():
        acc[...] = jnp.zeros((8, CH), jnp.float32)

    hmax, hmin, sh, sh2 = _edge_h(f_ref, g_ref, w_ref, b_ref)
    hmax_ref[...] = hmax
    hmin_ref[...] = hmin
    acc[...] = acc[...] + jnp.concatenate(
        [sh, sh2, jnp.zeros((6, CH), jnp.float32)], axis=0)

    @pl.when(i == NBLK - 1)
    def _():
        sums_ref[...] = acc[...]


def _edge_plain_body(f_ref, g_ref, w_ref, b_ref, fo_ref):
    hmax, _, _, _ = _edge_h(f_ref, g_ref, w_ref, b_ref)
    f = jnp.maximum(hmax, 0.0)
    fo_ref[...] = jnp.concatenate(
        [f, jnp.zeros((f.shape[0], FW - CH), jnp.float32)], axis=1)


def _edge_bn(f, g3, w, b):
    return pl.pallas_call(
        _edge_bn_body,
        grid=(NBLK,),
        in_specs=[
            pl.BlockSpec((ROWS, FW), lambda i: (i, 0)),
            pl.BlockSpec((ROWS, K, FW), lambda i: (i, 0, 0)),
            pl.BlockSpec((FW, CH), lambda i: (0, 0)),
            pl.BlockSpec((1, CH), lambda i: (0, 0)),
        ],
        out_specs=[
            pl.BlockSpec((ROWS, CH), lambda i: (i, 0)),
            pl.BlockSpec((ROWS, CH), lambda i: (i, 0)),
            pl.BlockSpec((8, CH), lambda i: (0, 0)),
        ],
        out_shape=[
            jax.ShapeDtypeStruct((N, CH), jnp.float32),
            jax.ShapeDtypeStruct((N, CH), jnp.float32),
            jax.ShapeDtypeStruct((8, CH), jnp.float32),
        ],
        scratch_shapes=[pltpu.VMEM((8, CH), jnp.float32)],
    )(f, g3, w, b)


def _edge_plain(f, g3, w, b):
    return pl.pallas_call(
        _edge_plain_body,
        grid=(NBLK,),
        in_specs=[
            pl.BlockSpec((ROWS, FW), lambda i: (i, 0)),
            pl.BlockSpec((ROWS, K, FW), lambda i: (i, 0, 0)),
            pl.BlockSpec((FW, CH), lambda i: (0, 0)),
            pl.BlockSpec((1, CH), lambda i: (0, 0)),
        ],
        out_specs=pl.BlockSpec((ROWS, FW), lambda i: (i, 0)),
        out_shape=jax.ShapeDtypeStruct((N, FW), jnp.float32),
    )(f, g3, w, b)


def _bnapply_body(hmax_ref, hmin_ref, sums_ref, g_ref, be_ref, fo_ref):
    s = sums_ref[...]
    inv_nk = 1.0 / (N * K)
    mean = s[0:1, :] * inv_nk
    var = s[1:2, :] * inv_nk - mean * mean
    g = g_ref[...]
    hsel = jnp.where(g >= 0.0, hmax_ref[...], hmin_ref[...])
    h = (hsel - mean) / jnp.sqrt(var + BN_EPS) * g + be_ref[...]
    f = jnp.maximum(h, 0.0)
    fo_ref[...] = jnp.concatenate(
        [f, jnp.zeros((f.shape[0], FW - CH), jnp.float32)], axis=1)


def _bnapply(hmax, hmin, sums, g, be):
    return pl.pallas_call(
        _bnapply_body,
        grid=(NBLK,),
        in_specs=[
            pl.BlockSpec((ROWS, CH), lambda i: (i, 0)),
            pl.BlockSpec((ROWS, CH), lambda i: (i, 0)),
            pl.BlockSpec((8, CH), lambda i: (0, 0)),
            pl.BlockSpec((1, CH), lambda i: (0, 0)),
            pl.BlockSpec((1, CH), lambda i: (0, 0)),
        ],
        out_specs=pl.BlockSpec((ROWS, FW), lambda i: (i, 0)),
        out_shape=jax.ShapeDtypeStruct((N, FW), jnp.float32),
    )(hmax, hmin, sums, g, be)


def _head_body(f1_ref, f2_ref, f3_ref, f4_ref, wp_ref, bp_ref, out_ref):
    parts = []
    for r in (f1_ref, f2_ref, f3_ref, f4_ref):
        f = r[...].reshape(B, N // B, FW)
        parts.append(jnp.max(f, axis=1)[:, :CH])
    a = jnp.concatenate(parts, axis=1)
    h = jnp.dot(a.astype(jnp.bfloat16),
                wp_ref[...].astype(jnp.bfloat16),
                preferred_element_type=jnp.float32)
    h = jnp.tanh(h + bp_ref[...])
    nrm = jnp.sqrt(jnp.sum(h * h, axis=1, keepdims=True))
    out_ref[...] = h / (nrm + EPS)


def _head(f1, f2, f3, f4, wp, bp):
    return pl.pallas_call(
        _head_body,
        grid=(1,),
        in_specs=[pl.BlockSpec((N, FW), lambda i: (0, 0))] * 4 + [
            pl.BlockSpec((4 * CH, OUT_DIMS), lambda i: (0, 0)),
            pl.BlockSpec((1, OUT_DIMS), lambda i: (0, 0)),
        ],
        out_specs=pl.BlockSpec((B, OUT_DIMS), lambda i: (0, 0)),
        out_shape=jax.ShapeDtypeStruct((B, OUT_DIMS), jnp.float32),
    )(f1, f2, f3, f4, wp, bp)


def _layer(f, w128, bvec, g=None, be=None):
    idxp = _knn(f.T, f)
    idx_flat = idxp[:, :K].reshape(-1)
    gath = _sc_gather(idx_flat, f).reshape(N, K, FW)
    if g is not None:
        hmax, hmin, sums = _edge_bn(f, gath, w128, bvec)
        return _bnapply(hmax, hmin, sums, g.reshape(1, CH), be.reshape(1, CH))
    return _edge_plain(f, gath, w128, bvec)


def _w128(w, d_in):
    out = jnp.zeros((FW, CH), jnp.float32)
    out = out.at[:d_in].set(w[:d_in])
    out = out.at[CH:CH + d_in].set(w[d_in:])
    return out


def kernel(x, n_pts, W1, b1, g1, be1, W2, b2, W3, b3, g3, be3, W4, b4, Wp, bp):
    del n_pts
    xp = jnp.pad(x, ((0, 0), (0, FW - x.shape[1])))

    f1 = _layer(xp, _w128(W1, 3), b1.reshape(1, CH), g1, be1)
    f2 = _layer(f1, _w128(W2, CH), b2.reshape(1, CH))
    f3 = _layer(f2, _w128(W3, CH), b3.reshape(1, CH), g3, be3)
    f4 = _layer(f3, _w128(W4, CH), b4.reshape(1, CH))

    return _head(f1, f2, f3, f4, Wp, bp.reshape(1, OUT_DIMS))
